# speedup vs baseline: 1.2748x; 1.2748x over previous
import functools

import jax
import jax.numpy as jnp
from jax import lax
from jax.experimental import pallas as pl
from jax.experimental.pallas import tpu as pltpu
from jax.experimental.pallas import tpu_sc as plsc

_EMBED = 128
_BLOCK = 2048
_BATCH = 4

_info = plsc.get_sparse_core_info()
_NC, _NS, _L = _info.num_cores, _info.num_subcores, _info.num_lanes
_NW = _NC * _NS
_B = _BATCH * _BLOCK
_BPW = _B // _NW
_CH = 128
_NCH = _BPW // _CH
_PROWS = _BLOCK // _CH


def _emb_body(x_hbm, tok_hbm, pos_hbm, out_hbm, idx_v, rows_v, pos_v, sem):
    wid = lax.axis_index("s") * _NC + lax.axis_index("c")
    rbase = wid * _NCH
    pbase = lax.rem(rbase, _PROWS)
    pltpu.sync_copy(x_hbm.at[pl.ds(rbase, _NCH)], idx_v)
    cps = [
        pltpu.async_copy(tok_hbm.at[idx_v.at[j]], rows_v.at[j], sem)
        for j in range(_NCH)
    ]
    pltpu.sync_copy(pos_hbm.at[pl.ds(pbase, _NCH)], pos_v)
    for cp in cps:
        cp.wait()

    def add_row(i, carry):
        for j in range(_NCH):
            for c in range(_EMBED // _L):
                s = pl.ds(c * _L, _L)
                rows_v[j, i, s] = rows_v[j, i, s] + pos_v[j, i, s]
        return carry

    lax.fori_loop(0, _CH, add_row, 0)
    pltpu.sync_copy(rows_v, out_hbm.at[pl.ds(rbase, _NCH)])


@jax.jit
def _emb(x_r, tok_table, pos_r):
    mesh = plsc.VectorSubcoreMesh(core_axis_name="c", subcore_axis_name="s")
    k = functools.partial(
        pl.kernel,
        mesh=mesh,
        out_type=jax.ShapeDtypeStruct((_B // _CH, _CH, _EMBED), jnp.float32),
        scratch_types=[
            pltpu.VMEM((_NCH, _CH), jnp.int32),
            pltpu.VMEM((_NCH, _CH, _EMBED), jnp.float32),
            pltpu.VMEM((_NCH, _CH, _EMBED), jnp.float32),
            pltpu.SemaphoreType.DMA,
        ],
    )(_emb_body)
    return k(x_r, tok_table, pos_r)


def kernel(x, tok_table, pos_table):
    x_r = x.astype(jnp.int32).reshape(_B // _CH, _CH)
    pos_r = pos_table.reshape(_PROWS, _CH, _EMBED)
    out = _emb(x_r, tok_table, pos_r)
    return out.reshape(_BATCH, _BLOCK, _EMBED)


# speedup vs baseline: 1.3109x; 1.0283x over previous
import functools

import jax
import jax.numpy as jnp
from jax import lax
from jax.experimental import pallas as pl
from jax.experimental.pallas import tpu as pltpu
from jax.experimental.pallas import tpu_sc as plsc

_EMBED = 128
_BLOCK = 2048
_BATCH = 4

_info = plsc.get_sparse_core_info()
_NC, _NS, _L = _info.num_cores, _info.num_subcores, _info.num_lanes
_NW = _NC * _NS
_SPW = _BLOCK // _NW


def _emb_body(x_hbm, tok_hbm, pos_hbm, out_hbm, idx_v, rows_v, pos_v,
              s0, s1, s2, s3, sp, so):
    gsems = [s0, s1, s2, s3]
    wid = lax.axis_index("s") * _NC + lax.axis_index("c")
    pcp = pltpu.async_copy(pos_hbm.at[wid], pos_v, sp)
    for b in range(_BATCH):
        pltpu.sync_copy(x_hbm.at[b * _NW + wid], idx_v.at[b])
    gcps = [
        pltpu.async_copy(tok_hbm.at[idx_v.at[b]], rows_v.at[b], gsems[b])
        for b in range(_BATCH)
    ]
    pcp.wait()
    ocps = []
    for b in range(_BATCH):
        gcps[b].wait()

        def add_row(i, carry, b=b):
            for c in range(_EMBED // _L):
                s = pl.ds(c * _L, _L)
                plsc.addupdate(rows_v.at[b, i, s], pos_v[i, s])
            return carry

        lax.fori_loop(0, _SPW, add_row, 0)
        ocps.append(
            pltpu.async_copy(rows_v.at[b], out_hbm.at[b * _NW + wid], so)
        )
    for cp in ocps:
        cp.wait()


@jax.jit
def _emb(x_r, tok_table, pos_r):
    mesh = plsc.VectorSubcoreMesh(core_axis_name="c", subcore_axis_name="s")
    k = functools.partial(
        pl.kernel,
        mesh=mesh,
        out_type=jax.ShapeDtypeStruct((_BATCH * _NW, _SPW, _EMBED),
                                      jnp.float32),
        scratch_types=[
            pltpu.VMEM((_BATCH, _SPW), jnp.int32),
            pltpu.VMEM((_BATCH, _SPW, _EMBED), jnp.float32),
            pltpu.VMEM((_SPW, _EMBED), jnp.float32),
        ] + [pltpu.SemaphoreType.DMA] * 6,
    )(_emb_body)
    return k(x_r, tok_table, pos_r)


def kernel(x, tok_table, pos_table):
    x_r = x.astype(jnp.int32).reshape(_BATCH * _NW, _SPW)
    pos_r = pos_table.reshape(_NW, _SPW, _EMBED)
    out = _emb(x_r, tok_table, pos_r)
    return out.reshape(_BATCH, _BLOCK, _EMBED)


# speedup vs baseline: 1.3828x; 1.0548x over previous
import functools

import jax
import jax.numpy as jnp
from jax import lax
from jax.experimental import pallas as pl
from jax.experimental.pallas import tpu as pltpu
from jax.experimental.pallas import tpu_sc as plsc

_EMBED = 128
_BLOCK = 2048
_BATCH = 4

_info = plsc.get_sparse_core_info()
_NC, _NS, _L = _info.num_cores, _info.num_subcores, _info.num_lanes
_NW = _NC * _NS
_SPW = _BLOCK // _NW


def _emb_body(x_hbm, tok_hbm, pos_hbm, out_hbm, idx_v, rows_v, pos_v,
              s0, s1, s2, s3, sp, so):
    gsems = [s0, s1, s2, s3]
    wid = lax.axis_index("s") * _NC + lax.axis_index("c")
    base = wid * _SPW
    pcp = pltpu.async_copy(pos_hbm.at[pl.ds(base, _SPW)], pos_v, sp)
    icps = [
        pltpu.async_copy(x_hbm.at[b, pl.ds(base, _SPW)], idx_v.at[b],
                         gsems[b])
        for b in range(_BATCH)
    ]
    gcps = []
    for b in range(_BATCH):
        icps[b].wait()
        gcps.append(
            pltpu.async_copy(tok_hbm.at[idx_v.at[b]], rows_v.at[b], gsems[b])
        )
    pcp.wait()
    ocps = []
    for b in range(_BATCH):
        gcps[b].wait()

        def add_row(i, carry, b=b):
            for c in range(_EMBED // _L):
                s = pl.ds(c * _L, _L)
                plsc.addupdate(rows_v.at[b, i, s], pos_v[i, s])
            return carry

        lax.fori_loop(0, _SPW, add_row, 0)
        ocps.append(
            pltpu.async_copy(rows_v.at[b], out_hbm.at[b, pl.ds(base, _SPW)],
                             so)
        )
    for cp in ocps:
        cp.wait()


@jax.jit
def _emb(x, tok_table, pos_table):
    mesh = plsc.VectorSubcoreMesh(core_axis_name="c", subcore_axis_name="s")
    k = functools.partial(
        pl.kernel,
        mesh=mesh,
        out_type=jax.ShapeDtypeStruct((_BATCH, _BLOCK, _EMBED), jnp.float32),
        scratch_types=[
            pltpu.VMEM((_BATCH, _SPW), jnp.int32),
            pltpu.VMEM((_BATCH, _SPW, _EMBED), jnp.float32),
            pltpu.VMEM((_SPW, _EMBED), jnp.float32),
        ] + [pltpu.SemaphoreType.DMA] * 6,
    )(_emb_body)
    return k(x, tok_table, pos_table)


def kernel(x, tok_table, pos_table):
    return _emb(x.astype(jnp.int32), tok_table, pos_table)
